# baseline (speedup 1.0000x reference)
import functools

import jax
import jax.numpy as jnp
from jax import lax
from jax.experimental import pallas as pl
from jax.experimental.pallas import tpu as pltpu
from jax.experimental.pallas import tpu_sc as plsc

_B, _N, _C = 32, 8732, 21
_FB = _N * _C
_T = _B * _FB
_ROWS_PER_CHUNK = 304
_CHUNK = _ROWS_PER_CHUNK * _C
_CHUNK_COPY = _CHUNK + 8
_NFULL = _N // _ROWS_PER_CHUNK
_TAIL_ROWS = _N - _NFULL * _ROWS_PER_CHUNK
_TAIL_ELEMS = _TAIL_ROWS * _C
_TAIL_COPY = 4624
_GROUPS_PER_CHUNK = _ROWS_PER_CHUNK // 16
_TAIL_FULL_GROUPS = _TAIL_ROWS // 16
_TAIL_REM_ROWS = _TAIL_ROWS - 16 * _TAIL_FULL_GROUPS

_LN2 = 0.6931471805599453
_P5 = (0.043928722358360296, -0.18983284334098535, 0.4115620160354525,
       -0.7072537269774657, 1.4415921402961083, 1.4387240710448608e-05)


def _log2_parts(x):
    b = lax.bitcast_convert_type(x, jnp.int32)
    e = lax.shift_right_arithmetic(b, 23)
    mb = lax.bitwise_or(lax.bitwise_and(b, 0x7FFFFF), 0x3F800000)
    m = lax.bitcast_convert_type(mb, jnp.float32)
    t = m - 1.0
    p = jnp.float32(_P5[0])
    for c in _P5[1:]:
        p = p * t + jnp.float32(c)
    return e, p


def _sc_body(lam_hbm, conf_hbm, shuf_hbm, interp_hbm, out_hbm,
             conf_v, shuf_v, interp_v, lam_v, stage_v):
    wid = lax.axis_index("s") * 2 + lax.axis_index("c")
    b = wid
    bs = jnp.bitwise_xor(wid, 16)
    base = b * _FB
    base_s = bs * _FB
    phase = 4 * jnp.bitwise_and(b, 1)

    pltpu.sync_copy(lam_hbm, lam_v)
    v_lam = lam_v[...]
    v_lamc = 1.0 - v_lam

    lanes = lax.iota(jnp.int32, 16)
    base_idx = lanes * _C + phase

    def group(goff, sum_acc, cnt_acc, lane_valid=None):
        idx0 = base_idx + goff
        acc = jnp.zeros((16,), jnp.float32)
        x0 = None
        t0 = None
        lmax = None
        rmax = None
        for c in range(_C):
            idx = idx0 + c
            x = plsc.load_gather(conf_v, [idx])
            t = plsc.load_gather(shuf_v, [idx])
            ci = plsc.load_gather(interp_v, [idx])
            m = v_lam * x + (v_lamc * t + 1e-7)
            i = ci + 1e-7
            d = i - m
            ei, pi_ = _log2_parts(i)
            em, pm = _log2_parts(m)
            ld2 = (ei - em).astype(jnp.float32) + (pi_ - pm)
            acc = acc + d * ld2
            if c == 0:
                x0, t0 = x, t
            elif c == 1:
                lmax, rmax = x, t
            else:
                lmax = jnp.maximum(lmax, x)
                rmax = jnp.maximum(rmax, t)
        gm = jnp.logical_and(lmax > x0, rmax > t0)
        if lane_valid is not None:
            gm = jnp.logical_and(gm, lane_valid)
        sum_acc = sum_acc + jnp.where(gm, acc, 0.0)
        cnt_acc = cnt_acc + jnp.where(gm, 1.0, 0.0)
        return sum_acc, cnt_acc

    def copy_chunk(elem_off, size):
        s0 = pl.multiple_of(base + elem_off - phase, 8)
        s1 = pl.multiple_of(base_s + elem_off - phase, 8)
        pltpu.sync_copy(conf_hbm.at[pl.ds(s0, size)],
                        conf_v.at[pl.ds(0, size)])
        pltpu.sync_copy(shuf_hbm.at[pl.ds(s1, size)],
                        shuf_v.at[pl.ds(0, size)])
        pltpu.sync_copy(interp_hbm.at[pl.ds(s0, size)],
                        interp_v.at[pl.ds(0, size)])

    def chunk_body(k, carry):
        sum_acc, cnt_acc = carry
        copy_chunk(k * _CHUNK, _CHUNK_COPY)

        def group_body(g, carry2):
            s, cn = carry2
            return group(g * (16 * _C), s, cn)

        return lax.fori_loop(0, _GROUPS_PER_CHUNK, group_body,
                             (sum_acc, cnt_acc))

    zeros = jnp.zeros((16,), jnp.float32)
    sum_acc, cnt_acc = lax.fori_loop(0, _NFULL, chunk_body, (zeros, zeros))

    copy_chunk(_NFULL * _CHUNK, _TAIL_COPY)

    def tail_group_body(g, carry2):
        s, cn = carry2
        return group(g * (16 * _C), s, cn)

    sum_acc, cnt_acc = lax.fori_loop(0, _TAIL_FULL_GROUPS, tail_group_body,
                                     (sum_acc, cnt_acc))
    valid = lanes < _TAIL_REM_ROWS
    sum_acc, cnt_acc = group(_TAIL_FULL_GROUPS * 16 * _C, sum_acc, cnt_acc,
                             lane_valid=valid)

    stage_v[pl.ds(0, 16)] = sum_acc * _LN2
    stage_v[pl.ds(16, 16)] = cnt_acc
    pltpu.sync_copy(stage_v, out_hbm.at[wid])


@functools.partial(pl.kernel,
                   out_type=jax.ShapeDtypeStruct((_B, 32), jnp.float32),
                   mesh=plsc.VectorSubcoreMesh(core_axis_name="c",
                                               subcore_axis_name="s"),
                   compiler_params=pltpu.CompilerParams(
                       use_tc_tiling_on_sc=False,
                       needs_layout_passes=False),
                   scratch_types=[
                       pltpu.VMEM((_CHUNK_COPY,), jnp.float32),
                       pltpu.VMEM((_CHUNK_COPY,), jnp.float32),
                       pltpu.VMEM((_CHUNK_COPY,), jnp.float32),
                       pltpu.VMEM((16,), jnp.float32),
                       pltpu.VMEM((32,), jnp.float32),
                   ])
def _sc_partials(lam_hbm, conf_hbm, shuf_hbm, interp_hbm, out_hbm,
                 conf_v, shuf_v, interp_v, lam_v, stage_v):
    _sc_body(lam_hbm, conf_hbm, shuf_hbm, interp_hbm, out_hbm,
             conf_v, shuf_v, interp_v, lam_v, stage_v)


def _finish_body(parts_ref, out_ref):
    p = parts_ref[...]
    s = jnp.sum(p[:, :16])
    c = jnp.sum(p[:, 16:])
    loss = 0.5 * s / jnp.maximum(c, 1.0)
    out_ref[0] = jnp.where(c > 0.0, loss, 0.0)


def _finish(parts):
    out = pl.pallas_call(
        _finish_body,
        out_specs=pl.BlockSpec(memory_space=pltpu.SMEM),
        out_shape=jax.ShapeDtypeStruct((1,), jnp.float32),
    )(parts)
    return out[0]


@jax.jit
def _isd_loss_sc(lam, conf, conf_shuffle, conf_interpolation):
    lam16 = jnp.full((16,), lam, jnp.float32)
    parts = _sc_partials(lam16,
                         conf.reshape(_T),
                         conf_shuffle.reshape(_T),
                         conf_interpolation.reshape(_T))
    return _finish(parts)


def kernel(lam, conf, conf_flip, loc, loc_flip, conf_shuffle,
           conf_interpolation, loc_shuffle, loc_interpolation):
    return _isd_loss_sc(lam, conf, conf_shuffle, conf_interpolation)


# speedup vs baseline: 1.9306x; 1.9306x over previous
import functools

import jax
import jax.numpy as jnp
from jax import lax
from jax.experimental import pallas as pl
from jax.experimental.pallas import tpu as pltpu
from jax.experimental.pallas import tpu_sc as plsc

_B, _N, _C = 32, 8732, 21
_NSC = 8728
_ROWS_PER_CHUNK = 208
_NFULL = _NSC // _ROWS_PER_CHUNK
_TAIL_ROWS = _NSC - _NFULL * _ROWS_PER_CHUNK
_GROUPS_PER_CHUNK = _ROWS_PER_CHUNK // 16
_TAIL_FULL_GROUPS = _TAIL_ROWS // 16
_TAIL_REM_ROWS = _TAIL_ROWS - 16 * _TAIL_FULL_GROUPS

_LN2 = 0.6931471805599453
_P5 = (0.043928722358360296, -0.18983284334098535, 0.4115620160354525,
       -0.7072537269774657, 1.4415921402961083, 1.4387240710448608e-05)


def _log2_parts(x):
    b = lax.bitcast_convert_type(x, jnp.int32)
    e = lax.shift_right_arithmetic(b, 23)
    mb = lax.bitwise_or(lax.bitwise_and(b, 0x7FFFFF), 0x3F800000)
    m = lax.bitcast_convert_type(mb, jnp.float32)
    t = m - 1.0
    p = jnp.float32(_P5[0])
    for c in _P5[1:]:
        p = p * t + jnp.float32(c)
    return e, p


def _sc_body(lam_hbm, conf_hbm, shuf_hbm, interp_hbm, out_hbm,
             conf_v, shuf_v, interp_v, lam_v, stage_v):
    wid = lax.axis_index("s") * 2 + lax.axis_index("c")
    b = wid
    bs = jnp.bitwise_xor(wid, 16)

    pltpu.sync_copy(lam_hbm, lam_v)
    v_lam = lam_v[...]
    v_lamc = 1.0 - v_lam

    lanes = lax.iota(jnp.int32, 16)

    def group(g, sum_acc, cnt_acc, lane_valid=None):
        rows = lanes + g * 16
        acc = jnp.zeros((16,), jnp.float32)
        x0 = None
        t0 = None
        lmax = None
        rmax = None
        for c in range(_C):
            cols = jnp.full((16,), c, jnp.int32)
            x = plsc.load_gather(conf_v, [rows, cols])
            t = plsc.load_gather(shuf_v, [rows, cols])
            ci = plsc.load_gather(interp_v, [rows, cols])
            m = v_lam * x + (v_lamc * t + 1e-7)
            i = ci + 1e-7
            d = i - m
            ei, pi_ = _log2_parts(i)
            em, pm = _log2_parts(m)
            ld2 = (ei - em).astype(jnp.float32) + (pi_ - pm)
            acc = acc + d * ld2
            if c == 0:
                x0, t0 = x, t
            elif c == 1:
                lmax, rmax = x, t
            else:
                lmax = jnp.maximum(lmax, x)
                rmax = jnp.maximum(rmax, t)
        gm = jnp.logical_and(lmax > x0, rmax > t0)
        if lane_valid is not None:
            gm = jnp.logical_and(gm, lane_valid)
        sum_acc = sum_acc + jnp.where(gm, acc, 0.0)
        cnt_acc = cnt_acc + jnp.where(gm, 1.0, 0.0)
        return sum_acc, cnt_acc

    def copy_chunk(row_off, nrows):
        r0 = pl.multiple_of(row_off, 8)
        pltpu.sync_copy(conf_hbm.at[b, pl.ds(r0, nrows), :],
                        conf_v.at[pl.ds(0, nrows), :])
        pltpu.sync_copy(shuf_hbm.at[bs, pl.ds(r0, nrows), :],
                        shuf_v.at[pl.ds(0, nrows), :])
        pltpu.sync_copy(interp_hbm.at[b, pl.ds(r0, nrows), :],
                        interp_v.at[pl.ds(0, nrows), :])

    def chunk_body(k, carry):
        sum_acc, cnt_acc = carry
        copy_chunk(k * _ROWS_PER_CHUNK, _ROWS_PER_CHUNK)

        def group_body(g, carry2):
            s, cn = carry2
            return group(g, s, cn)

        return lax.fori_loop(0, _GROUPS_PER_CHUNK, group_body,
                             (sum_acc, cnt_acc))

    zeros = jnp.zeros((16,), jnp.float32)
    sum_acc, cnt_acc = lax.fori_loop(0, _NFULL, chunk_body, (zeros, zeros))

    copy_chunk(_NFULL * _ROWS_PER_CHUNK, _TAIL_ROWS)

    def tail_group_body(g, carry2):
        s, cn = carry2
        return group(g, s, cn)

    sum_acc, cnt_acc = lax.fori_loop(0, _TAIL_FULL_GROUPS, tail_group_body,
                                     (sum_acc, cnt_acc))
    valid = lanes < _TAIL_REM_ROWS
    sum_acc, cnt_acc = group(_TAIL_FULL_GROUPS, sum_acc, cnt_acc,
                             lane_valid=valid)

    stage_v[pl.ds(0, 16)] = sum_acc * _LN2
    stage_v[pl.ds(16, 16)] = cnt_acc
    pltpu.sync_copy(stage_v, out_hbm.at[wid])


@functools.partial(pl.kernel,
                   out_type=jax.ShapeDtypeStruct((_B, 32), jnp.float32),
                   mesh=plsc.VectorSubcoreMesh(core_axis_name="c",
                                               subcore_axis_name="s"),
                   compiler_params=pltpu.CompilerParams(
                       use_tc_tiling_on_sc=True,
                       needs_layout_passes=False),
                   scratch_types=[
                       pltpu.VMEM((_ROWS_PER_CHUNK, _C), jnp.float32),
                       pltpu.VMEM((_ROWS_PER_CHUNK, _C), jnp.float32),
                       pltpu.VMEM((_ROWS_PER_CHUNK, _C), jnp.float32),
                       pltpu.VMEM((16,), jnp.float32),
                       pltpu.VMEM((32,), jnp.float32),
                   ])
def _sc_partials(lam_hbm, conf_hbm, shuf_hbm, interp_hbm, out_hbm,
                 conf_v, shuf_v, interp_v, lam_v, stage_v):
    _sc_body(lam_hbm, conf_hbm, shuf_hbm, interp_hbm, out_hbm,
             conf_v, shuf_v, interp_v, lam_v, stage_v)


def _finish_body(lam_ref, parts_ref, xt_ref, tt_ref, it_ref, out_ref):
    p = parts_ref[...]
    s = jnp.sum(p[:, :16])
    c = jnp.sum(p[:, 16:])

    lam = lam_ref[0]
    x = xt_ref[...]
    t = tt_ref[...]
    ci = it_ref[...]
    m = lam * x + (1.0 - lam) * t + 1e-7
    i = ci + 1e-7
    d = i - m
    ld = jnp.log(i) - jnp.log(m)
    kl = jnp.sum(d * ld, axis=-1)
    lmax = jnp.max(x[:, :, 1:], axis=2)
    rmax = jnp.max(t[:, :, 1:], axis=2)
    mf = jnp.logical_and(lmax > x[:, :, 0], rmax > t[:, :, 0]).astype(
        jnp.float32)
    s = s + jnp.sum(kl * mf)
    c = c + jnp.sum(mf)

    loss = 0.5 * s / jnp.maximum(c, 1.0)
    out_ref[0] = jnp.where(c > 0.0, loss, 0.0)


def _finish(lam, parts, x_tail, t_tail, i_tail):
    out = pl.pallas_call(
        _finish_body,
        in_specs=[
            pl.BlockSpec(memory_space=pltpu.SMEM),
            pl.BlockSpec(memory_space=pltpu.ANY
                         if False else pltpu.VMEM),
            pl.BlockSpec(memory_space=pltpu.VMEM),
            pl.BlockSpec(memory_space=pltpu.VMEM),
            pl.BlockSpec(memory_space=pltpu.VMEM),
        ],
        out_specs=pl.BlockSpec(memory_space=pltpu.SMEM),
        out_shape=jax.ShapeDtypeStruct((1,), jnp.float32),
    )(jnp.asarray(lam, jnp.float32).reshape(1), parts, x_tail, t_tail, i_tail)
    return out[0]


@jax.jit
def _isd_loss_sc(lam, conf, conf_shuffle, conf_interpolation):
    lam16 = jnp.full((16,), lam, jnp.float32)
    parts = _sc_partials(lam16, conf, conf_shuffle, conf_interpolation)
    x_tail = conf[:, _NSC:, :]
    ts = conf_shuffle[:, _NSC:, :]
    t_tail = jnp.concatenate([ts[_B // 2:], ts[:_B // 2]], axis=0)
    i_tail = conf_interpolation[:, _NSC:, :]
    return _finish(lam, parts, x_tail, t_tail, i_tail)


def kernel(lam, conf, conf_flip, loc, loc_flip, conf_shuffle,
           conf_interpolation, loc_shuffle, loc_interpolation):
    return _isd_loss_sc(lam, conf, conf_shuffle, conf_interpolation)


# speedup vs baseline: 46.6788x; 24.1790x over previous
import jax
import jax.numpy as jnp
from jax.experimental import pallas as pl
from jax.experimental.pallas import tpu as pltpu

_B, _N, _C = 32, 8732, 21


def _isd_body(lam_ref, x_ref, s_ref, i_ref, out_ref,
              acc_ref, x0_ref, t0_ref, lmax_ref, rmax_ref):
    c = pl.program_id(0)
    lam = lam_ref[0]

    x = x_ref[...]
    ts = s_ref[...]
    t = jnp.concatenate([ts[_B // 2:], ts[:_B // 2]], axis=0)
    ci = i_ref[...]

    mixed = lam * x + ((1.0 - lam) * t + 1e-7)
    interp = ci + 1e-7
    contrib = (interp - mixed) * (jnp.log(interp) - jnp.log(mixed))

    @pl.when(c == 0)
    def _():
        acc_ref[...] = contrib
        x0_ref[...] = x
        t0_ref[...] = t

    @pl.when(c > 0)
    def _():
        acc_ref[...] += contrib

    @pl.when(c == 1)
    def _():
        lmax_ref[...] = x
        rmax_ref[...] = t

    @pl.when(c > 1)
    def _():
        lmax_ref[...] = jnp.maximum(lmax_ref[...], x)
        rmax_ref[...] = jnp.maximum(rmax_ref[...], t)

    @pl.when(c == _C - 1)
    def _():
        mask = jnp.logical_and(lmax_ref[...] > x0_ref[...],
                               rmax_ref[...] > t0_ref[...])
        maskf = mask.astype(jnp.float32)
        s = jnp.sum(acc_ref[...] * maskf)
        cnt = jnp.sum(maskf)
        loss = 0.5 * s / jnp.maximum(cnt, 1.0)
        out_ref[0] = jnp.where(cnt > 0.0, loss, 0.0)


@jax.jit
def _isd_loss(lam, conf, conf_shuffle, conf_interpolation):
    ct = jnp.transpose(conf, (2, 0, 1)).reshape(_C * _B, _N)
    st = jnp.transpose(conf_shuffle, (2, 0, 1)).reshape(_C * _B, _N)
    it = jnp.transpose(conf_interpolation, (2, 0, 1)).reshape(_C * _B, _N)
    out = pl.pallas_call(
        _isd_body,
        grid=(_C,),
        in_specs=[
            pl.BlockSpec(memory_space=pltpu.SMEM),
            pl.BlockSpec((_B, _N), lambda c: (c, 0)),
            pl.BlockSpec((_B, _N), lambda c: (c, 0)),
            pl.BlockSpec((_B, _N), lambda c: (c, 0)),
        ],
        out_specs=pl.BlockSpec(memory_space=pltpu.SMEM),
        out_shape=jax.ShapeDtypeStruct((1,), jnp.float32),
        scratch_shapes=[
            pltpu.VMEM((_B, _N), jnp.float32),
            pltpu.VMEM((_B, _N), jnp.float32),
            pltpu.VMEM((_B, _N), jnp.float32),
            pltpu.VMEM((_B, _N), jnp.float32),
            pltpu.VMEM((_B, _N), jnp.float32),
        ],
        compiler_params=pltpu.CompilerParams(
            dimension_semantics=("arbitrary",),
        ),
    )(jnp.asarray(lam, jnp.float32).reshape(1), ct, st, it)
    return out[0]


def kernel(lam, conf, conf_flip, loc, loc_flip, conf_shuffle,
           conf_interpolation, loc_shuffle, loc_interpolation):
    return _isd_loss(lam, conf, conf_shuffle, conf_interpolation)


# speedup vs baseline: 54.1215x; 1.1594x over previous
import jax
import jax.numpy as jnp
from jax.experimental import pallas as pl
from jax.experimental.pallas import tpu as pltpu

_B, _N, _C = 32, 8732, 21
_CPB = 3
_STEPS = _C // _CPB


def _isd_body(lam_ref, x_ref, s_ref, i_ref, out_ref,
              acc_ref, x0_ref, t0_ref, lmax_ref, rmax_ref):
    g = pl.program_id(0)
    lam = lam_ref[0]
    half = _B // 2

    x = x_ref[...]
    ts = s_ref[...]
    ci = i_ref[...]
    t = jnp.concatenate(
        [ts[p * _B + (half if r == 0 else 0):
            p * _B + (_B if r == 0 else half)]
         for p in range(_CPB) for r in range(2)], axis=0)

    mixed = lam * x + ((1.0 - lam) * t + 1e-7)
    interp = ci + 1e-7
    contrib = (interp - mixed) * (jnp.log(interp) - jnp.log(mixed))

    csum = (contrib[:_B] + contrib[_B:2 * _B] + contrib[2 * _B:])
    m12 = jnp.maximum(x[_B:2 * _B], x[2 * _B:])
    r12 = jnp.maximum(t[_B:2 * _B], t[2 * _B:])

    @pl.when(g == 0)
    def _():
        acc_ref[...] = csum
        x0_ref[...] = x[:_B]
        t0_ref[...] = t[:_B]
        lmax_ref[...] = m12
        rmax_ref[...] = r12

    @pl.when(g > 0)
    def _():
        acc_ref[...] += csum
        lmax_ref[...] = jnp.maximum(lmax_ref[...], jnp.maximum(x[:_B], m12))
        rmax_ref[...] = jnp.maximum(rmax_ref[...], jnp.maximum(t[:_B], r12))

    @pl.when(g == _STEPS - 1)
    def _():
        mask = jnp.logical_and(lmax_ref[...] > x0_ref[...],
                               rmax_ref[...] > t0_ref[...])
        maskf = mask.astype(jnp.float32)
        s = jnp.sum(acc_ref[...] * maskf)
        cnt = jnp.sum(maskf)
        loss = 0.5 * s / jnp.maximum(cnt, 1.0)
        out_ref[0] = jnp.where(cnt > 0.0, loss, 0.0)


@jax.jit
def _isd_loss(lam, conf, conf_shuffle, conf_interpolation):
    ct = jnp.transpose(conf, (2, 0, 1)).reshape(_C * _B, _N)
    st = jnp.transpose(conf_shuffle, (2, 0, 1)).reshape(_C * _B, _N)
    it = jnp.transpose(conf_interpolation, (2, 0, 1)).reshape(_C * _B, _N)
    blk = _CPB * _B
    out = pl.pallas_call(
        _isd_body,
        grid=(_STEPS,),
        in_specs=[
            pl.BlockSpec(memory_space=pltpu.SMEM),
            pl.BlockSpec((blk, _N), lambda g: (g, 0)),
            pl.BlockSpec((blk, _N), lambda g: (g, 0)),
            pl.BlockSpec((blk, _N), lambda g: (g, 0)),
        ],
        out_specs=pl.BlockSpec(memory_space=pltpu.SMEM),
        out_shape=jax.ShapeDtypeStruct((1,), jnp.float32),
        scratch_shapes=[
            pltpu.VMEM((_B, _N), jnp.float32),
            pltpu.VMEM((_B, _N), jnp.float32),
            pltpu.VMEM((_B, _N), jnp.float32),
            pltpu.VMEM((_B, _N), jnp.float32),
            pltpu.VMEM((_B, _N), jnp.float32),
        ],
        compiler_params=pltpu.CompilerParams(
            dimension_semantics=("arbitrary",),
        ),
    )(jnp.asarray(lam, jnp.float32).reshape(1), ct, st, it)
    return out[0]


def kernel(lam, conf, conf_flip, loc, loc_flip, conf_shuffle,
           conf_interpolation, loc_shuffle, loc_interpolation):
    return _isd_loss(lam, conf, conf_shuffle, conf_interpolation)
